# baseline (speedup 1.0000x reference)
import functools

import jax
import jax.numpy as jnp
from jax.experimental import pallas as pl
from jax.experimental.pallas import tpu as pltpu

E = 8
TOPK = 2
D = 1024
F = 3584
T = 2048

BLK_T = 256
BLK_F = 512
NT = T // BLK_T
NF = F // BLK_F


def _router_body(x_ref, rw_ref, comb_ref):
    x = x_ref[...]
    logits = jax.lax.dot_general(
        x, rw_ref[...],
        (((1,), (1,)), ((), ())),
        preferred_element_type=jnp.float32,
    )
    m = jnp.max(logits, axis=-1, keepdims=True)
    p = jnp.exp(logits - m)
    p = p / jnp.sum(p, axis=-1, keepdims=True)
    lane = jax.lax.broadcasted_iota(jnp.int32, p.shape, 1)
    w1 = jnp.max(p, axis=-1, keepdims=True)
    i1 = jnp.min(jnp.where(p >= w1, lane, E), axis=-1, keepdims=True)
    p2 = jnp.where(lane == i1, -jnp.inf, p)
    w2 = jnp.max(p2, axis=-1, keepdims=True)
    i2 = jnp.min(jnp.where(p2 >= w2, lane, E), axis=-1, keepdims=True)
    s = w1 + w2
    comb = jnp.where(lane == i1, w1 / s, 0.0) + jnp.where(lane == i2, w2 / s, 0.0)
    comb_ref[...] = comb


def _moe_body(x_ref, comb_ref, wg_ref, wu_ref, wd_ref, out_ref):
    e = pl.program_id(0)
    f = pl.program_id(1)
    t = pl.program_id(2)

    @pl.when(jnp.logical_and(e == 0, f == 0))
    def _init():
        out_ref[pl.ds(t * BLK_T, BLK_T), :] = jnp.zeros(
            (BLK_T, D), dtype=jnp.float32)

    x = x_ref[pl.ds(t * BLK_T, BLK_T), :]
    g = jax.lax.dot_general(
        x, wg_ref[0], (((1,), (1,)), ((), ())),
        preferred_element_type=jnp.float32)
    u = jax.lax.dot_general(
        x, wu_ref[0], (((1,), (1,)), ((), ())),
        preferred_element_type=jnp.float32)
    h = (g * jax.lax.logistic(g) * u).astype(jnp.bfloat16)
    y = jax.lax.dot_general(
        h, wd_ref[0], (((1,), (1,)), ((), ())),
        preferred_element_type=jnp.float32)
    comb_blk = comb_ref[pl.ds(t * BLK_T, BLK_T), :]
    lane = jax.lax.broadcasted_iota(jnp.int32, comb_blk.shape, 1)
    scale = jnp.sum(jnp.where(lane == e, comb_blk, 0.0), axis=-1,
                    keepdims=True)
    out_ref[pl.ds(t * BLK_T, BLK_T), :] += scale * y


def kernel(hidden_states, router_weight, w_gate, w_up, w_down):
    b, s, d = hidden_states.shape
    x = hidden_states.reshape(-1, d)

    comb = pl.pallas_call(
        _router_body,
        out_shape=jax.ShapeDtypeStruct((T, E), jnp.float32),
        in_specs=[
            pl.BlockSpec((T, D), lambda: (0, 0)),
            pl.BlockSpec((E, D), lambda: (0, 0)),
        ],
        out_specs=pl.BlockSpec((T, E), lambda: (0, 0)),
    )(x, router_weight)

    xb = x.astype(jnp.bfloat16)
    wgb = w_gate.astype(jnp.bfloat16)
    wub = w_up.astype(jnp.bfloat16)
    wdb = w_down.astype(jnp.bfloat16)

    out = pl.pallas_call(
        _moe_body,
        grid=(E, NF, NT),
        out_shape=jax.ShapeDtypeStruct((T, D), jnp.float32),
        in_specs=[
            pl.BlockSpec((T, D), lambda e, f, t: (0, 0)),
            pl.BlockSpec((T, E), lambda e, f, t: (0, 0)),
            pl.BlockSpec((1, BLK_F, D), lambda e, f, t: (e, f, 0)),
            pl.BlockSpec((1, BLK_F, D), lambda e, f, t: (e, f, 0)),
            pl.BlockSpec((1, D, BLK_F), lambda e, f, t: (e, 0, f)),
        ],
        out_specs=pl.BlockSpec((T, D), lambda e, f, t: (0, 0)),
    )(xb, comb, wgb, wub, wdb)

    return out.reshape(b, s, d)


# speedup vs baseline: 1.0736x; 1.0736x over previous
import functools

import jax
import jax.numpy as jnp
from jax import lax
from jax.experimental import pallas as pl
from jax.experimental.pallas import tpu as pltpu
from jax.experimental.pallas import tpu_sc as plsc

E = 8
D = 1024
F = 3584
T = 2048

BLK = 256
NB = (2 * T) // BLK + E
P = NB * BLK
BLK_F = 512
NF = F // BLK_F

NW = 32
L = 16
TPW = T // NW
RPW = P // NW
NG = T // L


def _router_body(x_ref, rw_ref, logits_ref):
    logits_ref[...] = lax.dot_general(
        x_ref[...], rw_ref[...], (((1,), (1,)), ((), ())),
        preferred_element_type=jnp.float32)


def _dispatch_body(logits_hbm, x32_hbm,
                   xs_out, scale_out, pos0_out, pos1_out, be_out, bact_out,
                   logits_v, i1_v, i2_v, wa_v, wb_v,
                   own_idx, own_scale, rows_v,
                   p0_loc, p1_loc, be_v, bact_v, sem):
    w = lax.axis_index("c") * 16 + lax.axis_index("s")
    lanes = lax.iota(jnp.int32, L)

    pltpu.sync_copy(logits_hbm, logits_v)

    def pass_a(g, counts):
        base = g * L
        tok = base + lanes
        cols = [plsc.load_gather(logits_v, [tok * E + e]) for e in range(E)]
        m = cols[0]
        for e in range(1, E):
            m = jnp.maximum(m, cols[e])
        i1 = jnp.full((L,), E, jnp.int32)
        for e in range(E):
            i1 = jnp.minimum(i1, jnp.where(cols[e] == m,
                                           jnp.int32(e), jnp.int32(E)))
        m2 = jnp.full((L,), -jnp.inf, jnp.float32)
        cols2 = []
        for e in range(E):
            c2 = jnp.where(i1 == e, jnp.float32(-jnp.inf), cols[e])
            cols2.append(c2)
            m2 = jnp.maximum(m2, c2)
        i2 = jnp.full((L,), E, jnp.int32)
        for e in range(E):
            i2 = jnp.minimum(i2, jnp.where(cols2[e] == m2,
                                           jnp.int32(e), jnp.int32(E)))
        wa = 1.0 / (1.0 + jnp.exp(m2 - m))
        wb = 1.0 - wa
        sl = pl.ds(base, L)
        i1_v[sl] = i1
        i2_v[sl] = i2
        wa_v[sl] = wa
        wb_v[sl] = wb
        return tuple(
            counts[e]
            + plsc.all_reduce_population_count(i1 == e)
            + plsc.all_reduce_population_count(i2 == e)
            for e in range(E))

    zero = jnp.zeros((L,), jnp.int32)
    counts = lax.fori_loop(0, NG, pass_a, (zero,) * E)

    nb = [lax.shift_right_logical(counts[e] + (BLK - 1), 8) for e in range(E)]
    bstart = [zero]
    for e in range(E - 1):
        bstart.append(bstart[e] + nb[e])
    gstart = [b * BLK for b in bstart]
    total_nb = bstart[E - 1] + nb[E - 1]
    for k in range((NB + 8) // L):
        jv = lanes + k * L
        be = jnp.full((L,), -1, jnp.int32)
        for e in range(E):
            be = be + jnp.where(jv >= bstart[e], 1, 0)
        be_v[pl.ds(k * L, L)] = be
        bact_v[pl.ds(k * L, L)] = jnp.where(jv < total_nb, 1, 0)

    for c in range(2):
        for k in range(96 // L):
            own_idx[c, pl.ds(k * L, L)] = jnp.zeros((L,), jnp.int32)
    for k in range(RPW // L):
        own_scale[pl.ds(k * L, L)] = jnp.zeros((L,), jnp.float32)

    row_lo = w * RPW

    def pass_b(g, next_e):
        base = g * L
        sl = pl.ds(base, L)
        tok = base + lanes
        for which in range(2):
            ev = i1_v[sl] if which == 0 else i2_v[sl]
            wv = wa_v[sl] if which == 0 else wb_v[sl]
            basep = jnp.zeros((L,), jnp.int32)
            rank = jnp.zeros((L,), jnp.int32)
            nxt = []
            for e in range(E):
                mask = ev == e
                basep = jnp.where(mask, next_e[e], basep)
                cs = plsc.cumsum(mask.astype(jnp.int32))
                rank = rank + jnp.where(mask, cs - 1, 0)
                cnt = plsc.all_reduce_population_count(mask)
                nxt.append(next_e[e] + cnt)
            pos = basep + rank
            next_e = tuple(nxt)

            @pl.when(w == g // (TPW // L))
            def _():
                loc = pl.ds((g % (TPW // L)) * L, L)
                if which == 0:
                    p0_loc[loc] = pos
                else:
                    p1_loc[loc] = pos

            mrow = jnp.logical_and(pos >= row_lo, pos < row_lo + RPW)
            pl_ = jnp.where(mrow, pos - row_lo, 0)
            prow = jnp.where(pl_ >= 96, 1, 0)
            pcol = pl_ - prow * 96
            plsc.store_scatter(own_idx, [prow, pcol], tok, mask=mrow)
            plsc.store_scatter(own_scale, [pl_], wv, mask=mrow)
        return next_e

    lax.fori_loop(0, NG, pass_b, tuple(gstart))

    for c in range(2):
        pltpu.async_copy(x32_hbm.at[own_idx.at[c]],
                         rows_v.at[pl.ds(c * 96, 96)], sem).wait()

    pltpu.sync_copy(rows_v, xs_out.at[pl.ds(row_lo, RPW)])
    pltpu.sync_copy(own_scale, scale_out.at[pl.ds(row_lo, RPW)])
    pltpu.sync_copy(p0_loc, pos0_out.at[pl.ds(w * TPW, TPW)])
    pltpu.sync_copy(p1_loc, pos1_out.at[pl.ds(w * TPW, TPW)])

    @pl.when(w == 0)
    def _():
        pltpu.sync_copy(be_v, be_out)
        pltpu.sync_copy(bact_v, bact_out)


def _grouped_body(be_ref, bact_ref, x_ref, scale_ref,
                  wg_ref, wu_ref, wd_ref, out_ref):
    f = pl.program_id(0)
    i = pl.program_id(1)

    @pl.when(bact_ref[i] > 0)
    def _():
        rows = pl.ds(i * BLK, BLK)
        x = x_ref[rows, :]
        g = lax.dot_general(x, wg_ref[0], (((1,), (1,)), ((), ())),
                            preferred_element_type=jnp.float32)
        u = lax.dot_general(x, wu_ref[0], (((1,), (1,)), ((), ())),
                            preferred_element_type=jnp.float32)
        h = (g * lax.logistic(g) * u).astype(jnp.bfloat16)
        y = lax.dot_general(h, wd_ref[0], (((1,), (1,)), ((), ())),
                            preferred_element_type=jnp.float32)
        prev = jnp.where(f == 0, jnp.zeros((BLK, D), jnp.float32),
                         out_ref[rows, :])
        acc = prev + y
        acc = jnp.where(f == NF - 1, acc * scale_ref[rows, :], acc)
        out_ref[rows, :] = acc


def _combine_body(y_hbm, pos0_hbm, pos1_hbm, out_hbm,
                  idx0_v, idx1_v, a_v, b_v, sem0, sem1):
    w = lax.axis_index("c") * 16 + lax.axis_index("s")
    for c in range(2):
        base = w * TPW + c * 32
        pltpu.sync_copy(pos0_hbm.at[pl.ds(base, 32)], idx0_v)
        pltpu.sync_copy(pos1_hbm.at[pl.ds(base, 32)], idx1_v)
        cp_a = pltpu.async_copy(y_hbm.at[idx0_v], a_v, sem0)
        cp_b = pltpu.async_copy(y_hbm.at[idx1_v], b_v, sem1)
        cp_a.wait()
        cp_b.wait()

        def body(j, _):
            r = j // (D // L)
            sl = pl.ds((j % (D // L)) * L, L)
            a_v[r, sl] = a_v[r, sl] + b_v[r, sl]
            return 0

        lax.fori_loop(0, 32 * (D // L), body, 0)
        pltpu.sync_copy(a_v, out_hbm.at[pl.ds(base, 32)])


def kernel(hidden_states, router_weight, w_gate, w_up, w_down):
    b, s, d = hidden_states.shape
    x = hidden_states.reshape(-1, d)

    logits = pl.pallas_call(
        _router_body,
        out_shape=jax.ShapeDtypeStruct((T, E), jnp.float32),
        in_specs=[
            pl.BlockSpec((T, D), lambda: (0, 0)),
            pl.BlockSpec((E, D), lambda: (0, 0)),
        ],
        out_specs=pl.BlockSpec((T, E), lambda: (0, 0)),
    )(x, router_weight)

    xb = x.astype(jnp.bfloat16)
    x32 = lax.bitcast_convert_type(
        xb.reshape(T, D // 2, 2), jnp.float32)

    mesh = plsc.VectorSubcoreMesh(core_axis_name="c", subcore_axis_name="s")
    dispatch = pl.kernel(
        _dispatch_body,
        out_type=[
            jax.ShapeDtypeStruct((P, D // 2), jnp.float32),
            jax.ShapeDtypeStruct((P,), jnp.float32),
            jax.ShapeDtypeStruct((T,), jnp.int32),
            jax.ShapeDtypeStruct((T,), jnp.int32),
            jax.ShapeDtypeStruct((NB + 8,), jnp.int32),
            jax.ShapeDtypeStruct((NB + 8,), jnp.int32),
        ],
        mesh=mesh,
        scratch_types=[
            pltpu.VMEM((T * E,), jnp.float32),
            pltpu.VMEM((T,), jnp.int32),
            pltpu.VMEM((T,), jnp.int32),
            pltpu.VMEM((T,), jnp.float32),
            pltpu.VMEM((T,), jnp.float32),
            pltpu.VMEM((2, 96), jnp.int32),
            pltpu.VMEM((RPW,), jnp.float32),
            pltpu.VMEM((RPW, D // 2), jnp.float32),
            pltpu.VMEM((TPW,), jnp.int32),
            pltpu.VMEM((TPW,), jnp.int32),
            pltpu.VMEM((NB + 8,), jnp.int32),
            pltpu.VMEM((NB + 8,), jnp.int32),
            pltpu.SemaphoreType.DMA,
        ],
        compiler_params=pltpu.CompilerParams(needs_layout_passes=False),
    )
    xs32, scale, pos0, pos1, be, bact = dispatch(logits.reshape(T * E), x32)

    xs = lax.bitcast_convert_type(xs32, jnp.bfloat16).reshape(P, D)
    scale2 = scale.reshape(P, 1)
    wgb = w_gate.astype(jnp.bfloat16)
    wub = w_up.astype(jnp.bfloat16)
    wdb = w_down.astype(jnp.bfloat16)

    grid_spec = pltpu.PrefetchScalarGridSpec(
        num_scalar_prefetch=2,
        grid=(NF, NB),
        in_specs=[
            pl.BlockSpec((P, D), lambda f, i, be_r, ba_r: (0, 0)),
            pl.BlockSpec((P, 1), lambda f, i, be_r, ba_r: (0, 0)),
            pl.BlockSpec((1, BLK_F, D), lambda f, i, be_r, ba_r: (be_r[i], f, 0)),
            pl.BlockSpec((1, BLK_F, D), lambda f, i, be_r, ba_r: (be_r[i], f, 0)),
            pl.BlockSpec((1, D, BLK_F), lambda f, i, be_r, ba_r: (be_r[i], 0, f)),
        ],
        out_specs=pl.BlockSpec((P, D), lambda f, i, be_r, ba_r: (0, 0)),
    )
    y = pl.pallas_call(
        _grouped_body,
        grid_spec=grid_spec,
        out_shape=jax.ShapeDtypeStruct((P, D), jnp.float32),
    )(be, bact, xs, scale2, wgb, wub, wdb)

    combine = pl.kernel(
        _combine_body,
        out_type=jax.ShapeDtypeStruct((T, D), jnp.float32),
        mesh=mesh,
        scratch_types=[
            pltpu.VMEM((32,), jnp.int32),
            pltpu.VMEM((32,), jnp.int32),
            pltpu.VMEM((32, D), jnp.float32),
            pltpu.VMEM((32, D), jnp.float32),
            pltpu.SemaphoreType.DMA,
            pltpu.SemaphoreType.DMA,
        ],
        compiler_params=pltpu.CompilerParams(needs_layout_passes=False),
    )
    out = combine(y, pos0, pos1)
    return out.reshape(b, s, d)


# speedup vs baseline: 1.2667x; 1.1799x over previous
import functools

import jax
import jax.numpy as jnp
from jax import lax
from jax.experimental import pallas as pl
from jax.experimental.pallas import tpu as pltpu
from jax.experimental.pallas import tpu_sc as plsc

E = 8
D = 1024
F = 3584
T = 2048

BLK = 256
NB = (2 * T) // BLK + E
P = NB * BLK
BLK_F = 512
NF = F // BLK_F

NW = 32
L = 16
TPW = T // NW
RPW = P // NW
NG = T // L


def _router_body(x_ref, rw_ref, logits_ref):
    logits_ref[...] = lax.dot_general(
        x_ref[...], rw_ref[...], (((1,), (1,)), ((), ())),
        preferred_element_type=jnp.float32)


def _dispatch_body(logits_hbm, x32_hbm,
                   xs_out, scale_out, pos0_out, pos1_out, be_out, bact_out,
                   lg_v, i1l, i2l, wal, wbl, cnt1l, cnt2l,
                   sp_i1, sp_i2, sp_wa, sp_wb, sp_c1, sp_c2,
                   c1_v, c2_v, ev1, ev2, wa2, wb2,
                   rows_v, sA, sB, p0_loc, p1_loc, be_v, bact_v, sem):
    c_ax = lax.axis_index("c")
    sid = lax.axis_index("s")
    w = c_ax * 16 + sid
    lanes = lax.iota(jnp.int32, L)

    pltpu.sync_copy(logits_hbm.at[pl.ds(sid * 128 * E, 128 * E)], lg_v)
    for gl in range(8):
        tok = gl * L + lanes
        cols = [plsc.load_gather(lg_v, [tok * E + e]) for e in range(E)]
        m = cols[0]
        for e in range(1, E):
            m = jnp.maximum(m, cols[e])
        i1 = jnp.full((L,), E, jnp.int32)
        for e in range(E):
            i1 = jnp.minimum(i1, jnp.where(cols[e] == m,
                                           jnp.int32(e), jnp.int32(E)))
        m2 = jnp.full((L,), -jnp.inf, jnp.float32)
        cols2 = []
        for e in range(E):
            c2 = jnp.where(i1 == e, jnp.float32(-jnp.inf), cols[e])
            cols2.append(c2)
            m2 = jnp.maximum(m2, c2)
        i2 = jnp.full((L,), E, jnp.int32)
        for e in range(E):
            i2 = jnp.minimum(i2, jnp.where(cols2[e] == m2,
                                           jnp.int32(e), jnp.int32(E)))
        wa = 1.0 / (1.0 + jnp.exp(m2 - m))
        sl = pl.ds(gl * L, L)
        i1l[sl] = i1
        i2l[sl] = i2
        wal[sl] = wa
        wbl[sl] = 1.0 - wa
        cv1 = jnp.zeros((L,), jnp.int32)
        cv2 = jnp.zeros((L,), jnp.int32)
        for e in range(E):
            s1 = jnp.sum(jnp.where(i1 == e, 1, 0))
            s2 = jnp.sum(jnp.where(i2 == e, 1, 0))
            cv1 = cv1 + jnp.where(lanes == e, s1, 0)
            cv2 = cv2 + jnp.where(lanes == e, s2, 0)
        cnt1l[pl.ds(gl * L, L)] = cv1
        cnt2l[pl.ds(gl * L, L)] = cv2

    pltpu.sync_copy(i1l, sp_i1.at[sid])
    pltpu.sync_copy(i2l, sp_i2.at[sid])
    pltpu.sync_copy(wal, sp_wa.at[sid])
    pltpu.sync_copy(wbl, sp_wb.at[sid])
    pltpu.sync_copy(cnt1l, sp_c1.at[sid])
    pltpu.sync_copy(cnt2l, sp_c2.at[sid])
    plsc.subcore_barrier()

    pltpu.sync_copy(sp_c1, c1_v)
    pltpu.sync_copy(sp_c2, c2_v)
    pltpu.sync_copy(sp_i1.at[w // 2], ev1)
    pltpu.sync_copy(sp_i2.at[w // 2], ev2)
    pltpu.sync_copy(sp_wa.at[w // 2], wa2)
    pltpu.sync_copy(sp_wb.at[w // 2], wb2)
    toff = (w % 2) * 64

    def cnt_row(cv, r):
        return cv[r // 8, pl.ds((r % 8) * L, L)]

    def addrows(r, tot):
        return tot + cnt_row(c1_v, r) + cnt_row(c2_v, r)

    totals = jnp.zeros((L,), jnp.int32)
    for srow in range(16):
        for k in range(8):
            totals = (totals + c1_v[srow, pl.ds(k * L, L)]
                      + c2_v[srow, pl.ds(k * L, L)])
    nb = lax.shift_right_logical(totals + (BLK - 1), 8)
    bst = plsc.cumsum(nb) - nb
    gst = bst * BLK
    total_nb = jnp.sum(nb)
    for k in range((NB + 8) // L):
        jv = lanes + k * L
        be = jnp.full((L,), -1, jnp.int32)
        for e in range(E):
            bsp = jnp.sum(jnp.where(lanes == e, bst, 0))
            be = be + jnp.where(jv >= bsp, 1, 0)
        be_v[pl.ds(k * L, L)] = be
        bact_v[pl.ds(k * L, L)] = jnp.where(jv < total_nb, 1, 0)

    g0 = w * 4
    running = lax.fori_loop(0, g0, addrows, gst)
    for gl in range(4):
        for which in range(2):
            sl = pl.ds(toff + gl * L, L)
            ev = ev1[sl] if which == 0 else ev2[sl]
            base = jnp.zeros((L,), jnp.int32)
            rank = jnp.zeros((L,), jnp.int32)
            for e in range(E):
                mask = ev == e
                spl = jnp.sum(jnp.where(lanes == e, running, 0))
                base = jnp.where(mask, spl, base)
                cs = plsc.cumsum(mask.astype(jnp.int32))
                rank = rank + jnp.where(mask, cs - 1, 0)
            pos = base + rank
            osl = pl.ds(gl * L, L)
            if which == 0:
                p0_loc[osl] = pos
                running = running + cnt_row(c1_v, g0 + gl)
            else:
                p1_loc[osl] = pos
                running = running + cnt_row(c2_v, g0 + gl)

    for t in range(TPW):
        va = wa2[pl.ds(toff + (t // L) * L, L)]
        vb = wb2[pl.ds(toff + (t // L) * L, L)]
        spa = jnp.sum(jnp.where(lanes == (t % L), va, 0.0))
        spb = jnp.sum(jnp.where(lanes == (t % L), vb, 0.0))
        sA[t, pl.ds(0, L)] = jnp.full((L,), spa, jnp.float32)
        sB[t, pl.ds(0, L)] = jnp.full((L,), spb, jnp.float32)

    pltpu.sync_copy(x32_hbm.at[pl.ds(w * TPW, TPW)], rows_v)
    pltpu.async_copy(rows_v, xs_out.at[p0_loc], sem).wait()
    pltpu.async_copy(rows_v, xs_out.at[p1_loc], sem).wait()
    pltpu.async_copy(sA, scale_out.at[p0_loc], sem).wait()
    pltpu.async_copy(sB, scale_out.at[p1_loc], sem).wait()
    pltpu.sync_copy(p0_loc, pos0_out.at[pl.ds(w * TPW, TPW)])
    pltpu.sync_copy(p1_loc, pos1_out.at[pl.ds(w * TPW, TPW)])

    @pl.when(w == 0)
    def _():
        pltpu.sync_copy(be_v, be_out)
        pltpu.sync_copy(bact_v, bact_out)


def _grouped_body(be_ref, bact_ref, x_ref, scale_ref,
                  wg_ref, wu_ref, wd_ref, out_ref):
    f = pl.program_id(0)
    i = pl.program_id(1)

    @pl.when(bact_ref[i] > 0)
    def _():
        rows = pl.ds(i * BLK, BLK)
        x = x_ref[rows, :]
        g = lax.dot_general(x, wg_ref[0], (((1,), (1,)), ((), ())),
                            preferred_element_type=jnp.float32)
        u = lax.dot_general(x, wu_ref[0], (((1,), (1,)), ((), ())),
                            preferred_element_type=jnp.float32)
        h = (g * lax.logistic(g) * u).astype(jnp.bfloat16)
        y = lax.dot_general(h, wd_ref[0], (((1,), (1,)), ((), ())),
                            preferred_element_type=jnp.float32)
        prev = jnp.where(f == 0, jnp.zeros((BLK, D), jnp.float32),
                         out_ref[rows, :])
        acc = prev + y
        scale = scale_ref[rows, :][:, 0:1]
        acc = jnp.where(f == NF - 1, acc * scale, acc)
        out_ref[rows, :] = acc


def _combine_body(y_hbm, pos0_hbm, pos1_hbm, out_hbm,
                  idx0_v, idx1_v, a_v, b_v, sem0, sem1):
    w = lax.axis_index("c") * 16 + lax.axis_index("s")
    for c in range(2):
        base = w * TPW + c * 32
        pltpu.sync_copy(pos0_hbm.at[pl.ds(base, 32)], idx0_v)
        pltpu.sync_copy(pos1_hbm.at[pl.ds(base, 32)], idx1_v)
        cp_a = pltpu.async_copy(y_hbm.at[idx0_v], a_v, sem0)
        cp_b = pltpu.async_copy(y_hbm.at[idx1_v], b_v, sem1)
        cp_a.wait()
        cp_b.wait()

        def body(j, _):
            sl = pl.ds(j * L, L)
            for r in range(32):
                a_v[r, sl] = a_v[r, sl] + b_v[r, sl]
            return 0

        lax.fori_loop(0, D // L, body, 0)
        pltpu.sync_copy(a_v, out_hbm.at[pl.ds(base, 32)])


def kernel(hidden_states, router_weight, w_gate, w_up, w_down):
    b, s, d = hidden_states.shape
    x = hidden_states.reshape(-1, d)

    logits = pl.pallas_call(
        _router_body,
        out_shape=jax.ShapeDtypeStruct((T, E), jnp.float32),
        in_specs=[
            pl.BlockSpec((T, D), lambda: (0, 0)),
            pl.BlockSpec((E, D), lambda: (0, 0)),
        ],
        out_specs=pl.BlockSpec((T, E), lambda: (0, 0)),
    )(x, router_weight)

    xb = x.astype(jnp.bfloat16)
    x32 = lax.bitcast_convert_type(
        xb.reshape(T, D // 2, 2), jnp.float32)

    mesh = plsc.VectorSubcoreMesh(core_axis_name="c", subcore_axis_name="s")
    dispatch = pl.kernel(
        _dispatch_body,
        out_type=[
            jax.ShapeDtypeStruct((P, D // 2), jnp.float32),
            jax.ShapeDtypeStruct((P, 128), jnp.float32),
            jax.ShapeDtypeStruct((T,), jnp.int32),
            jax.ShapeDtypeStruct((T,), jnp.int32),
            jax.ShapeDtypeStruct((NB + 8,), jnp.int32),
            jax.ShapeDtypeStruct((NB + 8,), jnp.int32),
        ],
        mesh=mesh,
        scratch_types=[
            pltpu.VMEM((128 * E,), jnp.float32),
            pltpu.VMEM((128,), jnp.int32),
            pltpu.VMEM((128,), jnp.int32),
            pltpu.VMEM((128,), jnp.float32),
            pltpu.VMEM((128,), jnp.float32),
            pltpu.VMEM((128,), jnp.int32),
            pltpu.VMEM((128,), jnp.int32),
            pltpu.VMEM_SHARED((16, 128), jnp.int32),
            pltpu.VMEM_SHARED((16, 128), jnp.int32),
            pltpu.VMEM_SHARED((16, 128), jnp.float32),
            pltpu.VMEM_SHARED((16, 128), jnp.float32),
            pltpu.VMEM_SHARED((16, 128), jnp.int32),
            pltpu.VMEM_SHARED((16, 128), jnp.int32),
            pltpu.VMEM((16, 128), jnp.int32),
            pltpu.VMEM((16, 128), jnp.int32),
            pltpu.VMEM((128,), jnp.int32),
            pltpu.VMEM((128,), jnp.int32),
            pltpu.VMEM((128,), jnp.float32),
            pltpu.VMEM((128,), jnp.float32),
            pltpu.VMEM((TPW, D // 2), jnp.float32),
            pltpu.VMEM((TPW, 128), jnp.float32),
            pltpu.VMEM((TPW, 128), jnp.float32),
            pltpu.VMEM((TPW,), jnp.int32),
            pltpu.VMEM((TPW,), jnp.int32),
            pltpu.VMEM((NB + 8,), jnp.int32),
            pltpu.VMEM((NB + 8,), jnp.int32),
            pltpu.SemaphoreType.DMA,
        ],
        compiler_params=pltpu.CompilerParams(needs_layout_passes=False),
    )
    xs32, scale2, pos0, pos1, be, bact = dispatch(logits.reshape(T * E), x32)

    xs = lax.bitcast_convert_type(xs32, jnp.bfloat16).reshape(P, D)
    wgb = w_gate.astype(jnp.bfloat16)
    wub = w_up.astype(jnp.bfloat16)
    wdb = w_down.astype(jnp.bfloat16)

    grid_spec = pltpu.PrefetchScalarGridSpec(
        num_scalar_prefetch=2,
        grid=(NF, NB),
        in_specs=[
            pl.BlockSpec((P, D), lambda f, i, be_r, ba_r: (0, 0)),
            pl.BlockSpec((P, 128), lambda f, i, be_r, ba_r: (0, 0)),
            pl.BlockSpec((1, BLK_F, D), lambda f, i, be_r, ba_r: (be_r[i], f, 0)),
            pl.BlockSpec((1, BLK_F, D), lambda f, i, be_r, ba_r: (be_r[i], f, 0)),
            pl.BlockSpec((1, D, BLK_F), lambda f, i, be_r, ba_r: (be_r[i], 0, f)),
        ],
        out_specs=pl.BlockSpec((P, D), lambda f, i, be_r, ba_r: (0, 0)),
    )
    y = pl.pallas_call(
        _grouped_body,
        grid_spec=grid_spec,
        out_shape=jax.ShapeDtypeStruct((P, D), jnp.float32),
    )(be, bact, xs, scale2, wgb, wub, wdb)

    combine = pl.kernel(
        _combine_body,
        out_type=jax.ShapeDtypeStruct((T, D), jnp.float32),
        mesh=mesh,
        scratch_types=[
            pltpu.VMEM((32,), jnp.int32),
            pltpu.VMEM((32,), jnp.int32),
            pltpu.VMEM((32, D), jnp.float32),
            pltpu.VMEM((32, D), jnp.float32),
            pltpu.SemaphoreType.DMA,
            pltpu.SemaphoreType.DMA,
        ],
        compiler_params=pltpu.CompilerParams(needs_layout_passes=False),
    )
    out = combine(y, pos0, pos1)
    return out.reshape(b, s, d)
